# baseline (speedup 1.0000x reference)
import functools

import jax
import jax.numpy as jnp
from jax import lax
from jax.experimental import pallas as pl
from jax.experimental.pallas import tpu as pltpu
from jax.experimental.pallas import tpu_sc as plsc

H = 128
NEG = -1e30


def _emb_body(x_ref, w_ref, b_ref, o_ref):
    o_ref[...] = (
        jnp.dot(x_ref[...], w_ref[...], preferred_element_type=jnp.float32)
        + b_ref[...]
    )


def _emb(x, W, b, bs):
    n = x.shape[0]
    return pl.pallas_call(
        _emb_body,
        grid=(n // bs,),
        in_specs=[
            pl.BlockSpec((bs, H), lambda i: (i, 0)),
            pl.BlockSpec((H, H), lambda i: (0, 0)),
            pl.BlockSpec((1, H), lambda i: (0, 0)),
        ],
        out_specs=pl.BlockSpec((bs, H), lambda i: (i, 0)),
        out_shape=jax.ShapeDtypeStruct((n, H), jnp.float32),
    )(x, W, b.reshape(1, H))


def _sage_block(s, cnt, h, wl, bl, wr):
    mean = s * (1.0 / jnp.maximum(cnt, 1.0))
    out = (
        jnp.dot(mean, wl, preferred_element_type=jnp.float32)
        + bl
        + jnp.dot(h, wr, preferred_element_type=jnp.float32)
    )
    nrm = jnp.sqrt(jnp.sum(out * out, axis=-1, keepdims=True))
    return out / jnp.maximum(nrm, 1e-12)


def _layer_norm_block(t, g, b):
    mu = jnp.mean(t, axis=-1, keepdims=True)
    var = jnp.mean((t - mu) ** 2, axis=-1, keepdims=True)
    return (t - mu) / jnp.sqrt(var + 1e-5) * g + b


def _update2_body(s1_ref, c1_ref, s2_ref, c2_ref, h_ref,
                  wl1_ref, bl1_ref, wr1_ref, wl2_ref, bl2_ref, wr2_ref,
                  g_ref, bn_ref, o_ref):
    h = h_ref[...]
    o1 = _sage_block(s1_ref[...], c1_ref[...][:, 0:1], h,
                     wl1_ref[...], bl1_ref[...], wr1_ref[...])
    o2 = _sage_block(s2_ref[...], c2_ref[...][:, 0:1], h,
                     wl2_ref[...], bl2_ref[...], wr2_ref[...])
    t = jax.nn.relu((o1 + o2) * 0.5) + h
    o_ref[...] = _layer_norm_block(t, g_ref[...], bn_ref[...])


def _update1_body(s1_ref, c1_ref, h_ref, wl1_ref, bl1_ref, wr1_ref,
                  g_ref, bn_ref, o_ref):
    h = h_ref[...]
    o1 = _sage_block(s1_ref[...], c1_ref[...][:, 0:1], h,
                     wl1_ref[...], bl1_ref[...], wr1_ref[...])
    t = jax.nn.relu(o1) + h
    o_ref[...] = _layer_norm_block(t, g_ref[...], bn_ref[...])


def _row_spec(bs, w):
    return pl.BlockSpec((bs, w), lambda i: (i, 0))


def _full_spec(shape):
    return pl.BlockSpec(shape, lambda i: tuple(0 for _ in shape))


def _update2(s1, c1, s2, c2, h, wl1, bl1, wr1, wl2, bl2, wr2, g, bn, bs):
    n = h.shape[0]
    return pl.pallas_call(
        _update2_body,
        grid=(n // bs,),
        in_specs=[
            _row_spec(bs, H), _row_spec(bs, 16),
            _row_spec(bs, H), _row_spec(bs, 16),
            _row_spec(bs, H),
            _full_spec((H, H)), _full_spec((1, H)), _full_spec((H, H)),
            _full_spec((H, H)), _full_spec((1, H)), _full_spec((H, H)),
            _full_spec((1, H)), _full_spec((1, H)),
        ],
        out_specs=_row_spec(bs, H),
        out_shape=jax.ShapeDtypeStruct((n, H), jnp.float32),
    )(s1, c1, s2, c2, h, wl1, bl1.reshape(1, H), wr1,
      wl2, bl2.reshape(1, H), wr2, g.reshape(1, H), bn.reshape(1, H))


def _update1(s1, c1, h, wl1, bl1, wr1, g, bn, bs):
    n = h.shape[0]
    return pl.pallas_call(
        _update1_body,
        grid=(n // bs,),
        in_specs=[
            _row_spec(bs, H), _row_spec(bs, 16),
            _row_spec(bs, H),
            _full_spec((H, H)), _full_spec((1, H)), _full_spec((H, H)),
            _full_spec((1, H)), _full_spec((1, H)),
        ],
        out_specs=_row_spec(bs, H),
        out_shape=jax.ShapeDtypeStruct((n, H), jnp.float32),
    )(s1, c1, h, wl1, bl1.reshape(1, H), wr1, g.reshape(1, H), bn.reshape(1, H))


def _head_body(softmax, h_ref, w1_ref, b1_ref, w2_ref, b2_ref, o_ref):
    t = jax.nn.relu(
        jnp.dot(h_ref[...], w1_ref[...], preferred_element_type=jnp.float32)
        + b1_ref[...]
    )
    z = jnp.dot(t, w2_ref[...], preferred_element_type=jnp.float32) + b2_ref[...]
    if softmax:
        m = jnp.max(z, axis=-1, keepdims=True)
        z = z - m - jnp.log(jnp.sum(jnp.exp(z - m), axis=-1, keepdims=True))
    o_ref[...] = z


def _head(h, w1, b1, w2, b2, softmax, bs):
    n = h.shape[0]
    return pl.pallas_call(
        functools.partial(_head_body, softmax),
        grid=(n // bs,),
        in_specs=[
            _row_spec(bs, H),
            _full_spec((H, H)), _full_spec((1, H)),
            _full_spec((H, H)), _full_spec((1, H)),
        ],
        out_specs=_row_spec(bs, H),
        out_shape=jax.ShapeDtypeStruct((n, H), jnp.float32),
    )(h, w1, b1.reshape(1, H), w2, b2.reshape(1, H))


def _seg_mean_inputs(h_src, ei, n_dst, npad):
    src, dst = ei[0], ei[1]
    msg = jnp.take(h_src, src, axis=0)
    s = jax.ops.segment_sum(msg, dst, num_segments=n_dst)
    cnt = jax.ops.segment_sum(jnp.ones((src.shape[0],), jnp.float32), dst,
                              num_segments=n_dst)
    s = jnp.pad(s, ((0, npad - n_dst), (0, 0)))
    cnt = jnp.broadcast_to(cnt[:, None], (n_dst, 16))
    cnt = jnp.pad(cnt, ((0, npad - n_dst), (0, 0)))
    return s, cnt


def kernel(x_author, x_paper, params, edge_index_writes, edge_index_rev,
           edge_index_cites):
    p = params
    n_a = x_author.shape[0]
    n_p = x_paper.shape[0]
    bs = 1000 if n_a % 1000 == 0 else n_a
    npad_a = ((n_a + 12799) // 12800) * 12800
    npad_p = ((n_p + 12799) // 12800) * 12800

    h_a = _emb(x_author, p["W_emb_a"], p["b_emb_a"], bs)
    h_p = _emb(x_paper, p["W_emb_p"], p["b_emb_p"], bs)

    for l in range(2):
        s_w, c_w = _seg_mean_inputs(h_a, edge_index_writes, n_p, npad_p)
        s_r, c_r = _seg_mean_inputs(h_p, edge_index_rev, n_a, npad_a)
        s_c, c_c = _seg_mean_inputs(h_p, edge_index_cites, n_p, npad_p)
        new_a = _update1(s_r[:n_a], c_r[:n_a], h_a,
                         p[f"Wl{l}_rev"], p[f"bl{l}_rev"], p[f"Wr{l}_rev"],
                         p["ln_g_a"], p["ln_b_a"], bs)
        new_p = _update2(s_w[:n_p], c_w[:n_p], s_c[:n_p], c_c[:n_p], h_p,
                         p[f"Wl{l}_writes"], p[f"bl{l}_writes"],
                         p[f"Wr{l}_writes"],
                         p[f"Wl{l}_cites"], p[f"bl{l}_cites"],
                         p[f"Wr{l}_cites"],
                         p["ln_g_p"], p["ln_b_p"], bs)
        h_a, h_p = new_a, new_p

    c = p["Wo2_a"].shape[1]
    w2a = jnp.pad(p["Wo2_a"], ((0, 0), (0, H - c)))
    b2a = jnp.pad(p["bo2_a"], (0, H - c), constant_values=NEG)
    out_a = _head(h_a, p["Wo1_a"], p["bo1_a"], w2a, b2a, True, bs)[:, :c]
    out_p = _head(h_p, p["Wo1_p"], p["bo1_p"], p["Wo2_p"], p["bo2_p"],
                  False, bs)
    return (out_a, out_p)
